# baseline (speedup 1.0000x reference)
import jax
import jax.numpy as jnp
from jax import lax
from jax.experimental import pallas as pl
from jax.experimental.pallas import tpu as pltpu
from jax.experimental.pallas import tpu_sc as plsc

N = 10000
E = 320000
D = 128
HEAD = 4
DH = 32

NC = 2
NS = 16
NW = NC * NS
LANES = 16

N_P = 10240
ACC_R = 10016
BB = 128
NB = 80
BBA = 64
NBA = 160
EPT = NB * BB
E_P = NW * EPT
EROW = EPT + 80
TBLK = 256
TGRID = N_P // TBLK
RPT = ACC_R // NS

f32 = jnp.float32
i32 = jnp.int32


def _pre_body(x_ref, w_ref, asd_ref, zh_ref, alsd_ref):
    z = jnp.dot(x_ref[...], w_ref[...], preferred_element_type=f32)
    zh_ref[...] = z
    alsd_ref[...] = jnp.dot(z, asd_ref[...], preferred_element_type=f32)


def _tc_pre(x_p, W_att, ASD):
    return pl.pallas_call(
        _pre_body,
        grid=(TGRID,),
        in_specs=[pl.BlockSpec((TBLK, D), lambda i: (i, 0)),
                  pl.BlockSpec((D, D), lambda i: (0, 0)),
                  pl.BlockSpec((D, 16), lambda i: (0, 0))],
        out_specs=[pl.BlockSpec((TBLK, D), lambda i: (i, 0)),
                   pl.BlockSpec((TBLK, 16), lambda i: (i, 0))],
        out_shape=[jax.ShapeDtypeStruct((N_P, D), f32),
                   jax.ShapeDtypeStruct((N_P, 16), f32)],
    )(x_p, W_att, ASD)


def _comb1_body(a0_ref, a1_ref, b0_ref, b1_ref, rs_ref, rd_ref, w_ref,
                h_ref, nrm_ref, g_ref):
    num = a0_ref[...] + a1_ref[...]
    tails = b0_ref[...] + b1_ref[...]
    sexp_b = jnp.dot(tails, rs_ref[...], preferred_element_type=f32)
    deg_b = jnp.dot(tails, rd_ref[...], preferred_element_type=f32) + 1.0
    z = num / (sexp_b + 1e-16)
    rid = pl.program_id(0) * TBLK + lax.broadcasted_iota(i32, (TBLK, D), 0)
    h = jnp.where(rid < N, jnp.maximum(z, 0.0), 0.0)
    nrm = lax.rsqrt(deg_b)
    h_ref[...] = h
    nrm_ref[...] = nrm
    g_ref[...] = jnp.dot(h, w_ref[...], preferred_element_type=f32) * nrm


def _tc_comb1(acc0, acc1, t0, t1, RS, RD, W1):
    return pl.pallas_call(
        _comb1_body,
        grid=(TGRID,),
        in_specs=[pl.BlockSpec((TBLK, D), lambda i: (i, 0)),
                  pl.BlockSpec((TBLK, D), lambda i: (i, 0)),
                  pl.BlockSpec((TBLK, 16), lambda i: (i, 0)),
                  pl.BlockSpec((TBLK, 16), lambda i: (i, 0)),
                  pl.BlockSpec((16, D), lambda i: (0, 0)),
                  pl.BlockSpec((16, D), lambda i: (0, 0)),
                  pl.BlockSpec((D, D), lambda i: (0, 0))],
        out_specs=[pl.BlockSpec((TBLK, D), lambda i: (i, 0)),
                   pl.BlockSpec((TBLK, D), lambda i: (i, 0)),
                   pl.BlockSpec((TBLK, D), lambda i: (i, 0))],
        out_shape=[jax.ShapeDtypeStruct((N_P, D), f32),
                   jax.ShapeDtypeStruct((N_P, D), f32),
                   jax.ShapeDtypeStruct((N_P, D), f32)],
    )(acc0, acc1, t0, t1, RS, RD, W1)


def _layer_body(h_ref, g_ref, nrm_ref, a0_ref, a1_ref, w_ref, b_ref,
                hn_ref, gn_ref):
    nrm = nrm_ref[...]
    o = nrm * (a0_ref[...] + a1_ref[...] + g_ref[...]) + b_ref[...]
    rid = pl.program_id(0) * TBLK + lax.broadcasted_iota(i32, (TBLK, D), 0)
    hn = h_ref[...] + jnp.where(rid < N, jnp.maximum(o, 0.0), 0.0)
    hn_ref[...] = hn
    gn_ref[...] = jnp.dot(hn, w_ref[...], preferred_element_type=f32) * nrm


def _tc_layer(h, g, nrm, acc0, acc1, Wn, b):
    return pl.pallas_call(
        _layer_body,
        grid=(TGRID,),
        in_specs=[pl.BlockSpec((TBLK, D), lambda i: (i, 0)),
                  pl.BlockSpec((TBLK, D), lambda i: (i, 0)),
                  pl.BlockSpec((TBLK, D), lambda i: (i, 0)),
                  pl.BlockSpec((TBLK, D), lambda i: (i, 0)),
                  pl.BlockSpec((TBLK, D), lambda i: (i, 0)),
                  pl.BlockSpec((D, D), lambda i: (0, 0)),
                  pl.BlockSpec((1, D), lambda i: (0, 0))],
        out_specs=[pl.BlockSpec((TBLK, D), lambda i: (i, 0)),
                   pl.BlockSpec((TBLK, D), lambda i: (i, 0))],
        out_shape=[jax.ShapeDtypeStruct((N_P, D), f32),
                   jax.ShapeDtypeStruct((N_P, D), f32)],
    )(h, g, nrm, acc0, acc1, Wn, b)


def _final_body(g_ref, nrm_ref, a0_ref, a1_ref, b_ref, o_ref):
    o_ref[...] = (nrm_ref[...] * (a0_ref[...] + a1_ref[...] + g_ref[...])
                  + b_ref[...])


def _tc_final(g, nrm, acc0, acc1, b):
    return pl.pallas_call(
        _final_body,
        grid=(TGRID,),
        in_specs=[pl.BlockSpec((TBLK, D), lambda i: (i, 0)),
                  pl.BlockSpec((TBLK, D), lambda i: (i, 0)),
                  pl.BlockSpec((TBLK, D), lambda i: (i, 0)),
                  pl.BlockSpec((TBLK, D), lambda i: (i, 0)),
                  pl.BlockSpec((1, D), lambda i: (0, 0))],
        out_specs=pl.BlockSpec((TBLK, D), lambda i: (i, 0)),
        out_shape=jax.ShapeDtypeStruct((N_P, D), f32),
    )(g, nrm, acc0, acc1, b)


_SC_MESH = plsc.VectorSubcoreMesh(core_axis_name="c", subcore_axis_name="s",
                                  num_cores=NC, num_subcores=NS)


def _acc_init(zr_hbm, bounce, acc, s, cb):
    pltpu.sync_copy(zr_hbm.at[pl.ds(0, cb)], bounce)
    base = s * RPT
    nfull, rem = divmod(RPT, cb)
    for k in range(nfull):
        pltpu.sync_copy(bounce, acc.at[pl.ds(base + k * cb, cb)])
    if rem:
        pltpu.sync_copy(bounce.at[pl.ds(0, rem)],
                        acc.at[pl.ds(base + nfull * cb, rem)])
    plsc.subcore_barrier()


def _acc_writeback(acc, bounce, out, c, s, cb):
    plsc.subcore_barrier()
    base = s * RPT
    nfull, rem = divmod(RPT, cb)
    for k in range(nfull):
        r0 = base + k * cb
        pltpu.sync_copy(acc.at[pl.ds(r0, cb)], bounce)
        pltpu.sync_copy(bounce, out.at[c, pl.ds(r0, cb)])
    if rem:
        r0 = base + nfull * cb
        pltpu.sync_copy(acc.at[pl.ds(r0, rem)], bounce.at[pl.ds(0, rem)])
        pltpu.sync_copy(bounce.at[pl.ds(0, rem)], out.at[c, pl.ds(r0, rem)])


def _edge_w(srows, drows, i):
    e = srows[i] + drows[i]
    e = jnp.maximum(e, 0.2 * e)
    return jnp.exp(e)


def _att_edges(rows, srows, drows):
    def edge(i, carry2):
        w = _edge_w(srows, drows, i)
        for h in range(HEAD):
            wv = jnp.full((LANES,), w[h], f32)
            for k2 in range(2):
                col = h * DH + k2 * LANES
                rows[i, pl.ds(col, LANES)] = rows[i, pl.ds(col, LANES)] * wv
        return carry2

    lax.fori_loop(0, BBA, edge, 0)


def _attf_sc_body(srcp3, dstp3, zh_t, als_t, ald_t, zr_hbm, out,
                  gi, di, ra, rb, sra, srb, dra, drb, acc, sem):
    c = lax.axis_index("c")
    s = lax.axis_index("s")
    wid = c * NS + s
    _acc_init(zr_hbm, ra, acc, s, BBA)
    pltpu.sync_copy(srcp3.at[wid], gi)
    pltpu.sync_copy(dstp3.at[wid], di)

    def issue(j, rr, sr, dr):
        pltpu.async_copy(zh_t.at[gi.at[j]], rr, sem)
        pltpu.async_copy(als_t.at[gi.at[j]], sr, sem)
        pltpu.async_copy(ald_t.at[di.at[j]], dr, sem)

    def waitg(j, rr, sr, dr):
        pltpu.make_async_copy(zh_t.at[gi.at[j]], rr, sem).wait()
        pltpu.make_async_copy(als_t.at[gi.at[j]], sr, sem).wait()
        pltpu.make_async_copy(ald_t.at[di.at[j]], dr, sem).wait()

    issue(0, ra, sra, dra)
    issue(1, rb, srb, drb)

    def pair(p, carry):
        j0 = 2 * p
        for j, rr, sr, dr in ((j0, ra, sra, dra), (j0 + 1, rb, srb, drb)):
            waitg(j, rr, sr, dr)
            _att_edges(rr, sr, dr)
            pltpu.sync_copy(rr, acc.at[di.at[j]], add=True)

            @pl.when(j + 2 < NBA)
            def _():
                issue(j + 2, rr, sr, dr)
        return carry

    lax.fori_loop(0, NBA // 2, pair, 0)
    _acc_writeback(acc, ra, out, c, s, BBA)


_attf_sc = pl.kernel(
    _attf_sc_body,
    out_type=jax.ShapeDtypeStruct((NC, ACC_R, D), f32),
    mesh=_SC_MESH,
    compiler_params=pltpu.CompilerParams(use_tc_tiling_on_sc=False),
    scratch_types=[
        pltpu.VMEM((NBA, BBA), i32),
        pltpu.VMEM((NBA, BBA), i32),
        pltpu.VMEM((BBA, D), f32),
        pltpu.VMEM((BBA, D), f32),
        pltpu.VMEM((BBA, LANES), f32),
        pltpu.VMEM((BBA, LANES), f32),
        pltpu.VMEM((BBA, LANES), f32),
        pltpu.VMEM((BBA, LANES), f32),
        pltpu.VMEM_SHARED((ACC_R, D), f32),
        pltpu.SemaphoreType.DMA,
    ],
)


def _attt_sc_body(srcp3, dstp3, als_t, ald_t, zr16_hbm, out,
                  gi, di, ta, tb, sra, srb, dra, drb, acc, sem):
    c = lax.axis_index("c")
    s = lax.axis_index("s")
    wid = c * NS + s
    _acc_init(zr16_hbm, ta, acc, s, BBA)
    pltpu.sync_copy(srcp3.at[wid], gi)
    pltpu.sync_copy(dstp3.at[wid], di)
    iot = lax.iota(i32, LANES)
    tail_one = jnp.where(iot == HEAD, 1.0, 0.0).astype(f32)

    def tails(trows, srows, drows):
        def edge(i, carry2):
            w = _edge_w(srows, drows, i)
            trows[i] = jnp.where(iot < HEAD, w, tail_one)
            return carry2

        lax.fori_loop(0, BBA, edge, 0)

    def issue(j, sr, dr):
        pltpu.async_copy(als_t.at[gi.at[j]], sr, sem)
        pltpu.async_copy(ald_t.at[di.at[j]], dr, sem)

    def waitg(j, sr, dr):
        pltpu.make_async_copy(als_t.at[gi.at[j]], sr, sem).wait()
        pltpu.make_async_copy(ald_t.at[di.at[j]], dr, sem).wait()

    issue(0, sra, dra)
    issue(1, srb, drb)

    def pair(p, carry):
        j0 = 2 * p
        for j, tt, sr, dr in ((j0, ta, sra, dra), (j0 + 1, tb, srb, drb)):
            waitg(j, sr, dr)
            tails(tt, sr, dr)
            pltpu.sync_copy(tt, acc.at[di.at[j]], add=True)

            @pl.when(j + 2 < NBA)
            def _():
                issue(j + 2, sr, dr)
        return carry

    lax.fori_loop(0, NBA // 2, pair, 0)
    _acc_writeback(acc, ta, out, c, s, BBA)


_attt_sc = pl.kernel(
    _attt_sc_body,
    out_type=jax.ShapeDtypeStruct((NC, ACC_R, LANES), f32),
    mesh=_SC_MESH,
    compiler_params=pltpu.CompilerParams(use_tc_tiling_on_sc=False),
    scratch_types=[
        pltpu.VMEM((NBA, BBA), i32),
        pltpu.VMEM((NBA, BBA), i32),
        pltpu.VMEM((BBA, LANES), f32),
        pltpu.VMEM((BBA, LANES), f32),
        pltpu.VMEM((BBA, LANES), f32),
        pltpu.VMEM((BBA, LANES), f32),
        pltpu.VMEM((BBA, LANES), f32),
        pltpu.VMEM((BBA, LANES), f32),
        pltpu.VMEM_SHARED((ACC_R, LANES), f32),
        pltpu.SemaphoreType.DMA,
    ],
)


def _gcn_sc_body(srcp3, dstp3, g_t, zr_hbm, out, idx_s, dia, dib, ra, rb,
                 acc, sem):
    c = lax.axis_index("c")
    s = lax.axis_index("s")
    wid = c * NS + s
    _acc_init(zr_hbm, ra, acc, s, BB)
    pltpu.sync_copy(srcp3.at[wid], idx_s)

    def issue(j, rr, dd):
        pltpu.async_copy(g_t.at[idx_s.at[j]], rr, sem)
        pltpu.async_copy(dstp3.at[wid, j], dd, sem)

    def waitg(j, rr, dd):
        pltpu.make_async_copy(g_t.at[idx_s.at[j]], rr, sem).wait()
        pltpu.make_async_copy(dstp3.at[wid, j], dd, sem).wait()

    issue(0, ra, dia)
    issue(1, rb, dib)

    def pair(p, carry):
        j0 = 2 * p
        for j, rr, dd in ((j0, ra, dia), (j0 + 1, rb, dib)):
            waitg(j, rr, dd)
            pltpu.sync_copy(rr, acc.at[dd], add=True)

            @pl.when(j + 2 < NB)
            def _():
                issue(j + 2, rr, dd)
        return carry

    lax.fori_loop(0, NB // 2, pair, 0)
    _acc_writeback(acc, ra, out, c, s, BB)


_gcn_sc = pl.kernel(
    _gcn_sc_body,
    out_type=jax.ShapeDtypeStruct((NC, ACC_R, D), f32),
    mesh=_SC_MESH,
    compiler_params=pltpu.CompilerParams(use_tc_tiling_on_sc=False),
    scratch_types=[
        pltpu.VMEM((NB, BB), i32),
        pltpu.VMEM((BB,), i32),
        pltpu.VMEM((BB,), i32),
        pltpu.VMEM((BB, D), f32),
        pltpu.VMEM((BB, D), f32),
        pltpu.VMEM_SHARED((ACC_R, D), f32),
        pltpu.SemaphoreType.DMA,
    ],
)


def kernel(x, edge_index, W_att, a_src, a_dst,
           W1, b1, W2, b2, W3, b3, W4, b4, W5, b5):
    x_p = jnp.concatenate([x, jnp.zeros((N_P - N, D), f32)], axis=0)
    padt = jnp.full((NW, EROW - EPT), N, i32)
    src2 = jnp.concatenate(
        [jnp.concatenate([edge_index[0], jnp.full((E_P - E,), N, i32)])
         .reshape(NW, EPT), padt], axis=1)
    dst2 = jnp.concatenate(
        [jnp.concatenate([edge_index[1], jnp.full((E_P - E,), N, i32)])
         .reshape(NW, EPT), padt], axis=1)
    src3 = src2[:, :EPT].reshape(NW, NB, BB)
    dst3 = dst2[:, :EPT].reshape(NW, NB, BB)
    src3a = src2[:, :EPT].reshape(NW, NBA, BBA)
    dst3a = dst2[:, :EPT].reshape(NW, NBA, BBA)

    karr = jnp.arange(D)
    h16 = jnp.arange(16)
    mask_s = ((karr[:, None] // DH == h16[None, :]) &
              (h16[None, :] < HEAD)).astype(f32)
    mask_d = ((karr[:, None] // DH == h16[None, :] - HEAD) &
              (h16[None, :] >= HEAD) & (h16[None, :] < 2 * HEAD)).astype(f32)
    ASD = (a_src.reshape(D)[:, None] * mask_s
           + a_dst.reshape(D)[:, None] * mask_d)
    RS = ((h16[:, None] < HEAD) &
          (h16[:, None] == karr[None, :] // DH)).astype(f32)
    RD = (h16[:, None] == HEAD).astype(f32) * jnp.ones((1, D), f32)

    zh_t, alsd16 = _tc_pre(x_p, W_att, ASD)
    als_t = alsd16
    ald_t = jnp.roll(alsd16, -HEAD, axis=1)

    zr = jnp.zeros((BB, D), f32)
    zr16 = jnp.zeros((BBA, 16), f32)
    apad = jnp.zeros((N_P - ACC_R, D), f32)
    apad16 = jnp.zeros((N_P - ACC_R, 16), f32)

    def padacc(a):
        return jnp.concatenate([a, apad], axis=0)

    def padacc16(a):
        return jnp.concatenate([a, apad16], axis=0)

    accf = _attf_sc(src3a, dst3a, zh_t, als_t, ald_t, zr)
    acct = _attt_sc(src3a, dst3a, als_t, ald_t, zr16)
    h, nrm, g = _tc_comb1(padacc(accf[0]), padacc(accf[1]),
                          padacc16(acct[0]), padacc16(acct[1]), RS, RD, W1)

    for Wn, b in ((W2, b1), (W3, b2), (W4, b3), (W5, b4)):
        accg = _gcn_sc(src3, dst3, g, zr)
        h, g = _tc_layer(h, g, nrm, padacc(accg[0]), padacc(accg[1]),
                         Wn, b.reshape(1, D))
    accg = _gcn_sc(src3, dst3, g, zr)
    out = _tc_final(g, nrm, padacc(accg[0]), padacc(accg[1]),
                    b5.reshape(1, D))
    return out[:N]


# speedup vs baseline: 1.0339x; 1.0339x over previous
import jax
import jax.numpy as jnp
from jax import lax
from jax.experimental import pallas as pl
from jax.experimental.pallas import tpu as pltpu
from jax.experimental.pallas import tpu_sc as plsc

N = 10000
E = 320000
D = 128
HEAD = 4
DH = 32

NC = 2
NS = 16
NW = NC * NS
LANES = 16

N_P = 10240
ACC_R = 10016
BB = 128
NB = 80
BBA = 128
NBA = 80
EPT = NB * BB
E_P = NW * EPT
EROW = EPT + 80
TBLK = 256
TGRID = N_P // TBLK
RPT = ACC_R // NS

f32 = jnp.float32
i32 = jnp.int32


def _pre_body(x_ref, w_ref, asd_ref, zh_ref, alsd_ref):
    z = jnp.dot(x_ref[...], w_ref[...], preferred_element_type=f32)
    zh_ref[...] = z
    alsd_ref[...] = jnp.dot(z, asd_ref[...], preferred_element_type=f32)


def _tc_pre(x_p, W_att, ASD):
    return pl.pallas_call(
        _pre_body,
        grid=(TGRID,),
        in_specs=[pl.BlockSpec((TBLK, D), lambda i: (i, 0)),
                  pl.BlockSpec((D, D), lambda i: (0, 0)),
                  pl.BlockSpec((D, 16), lambda i: (0, 0))],
        out_specs=[pl.BlockSpec((TBLK, D), lambda i: (i, 0)),
                   pl.BlockSpec((TBLK, 16), lambda i: (i, 0))],
        out_shape=[jax.ShapeDtypeStruct((N_P, D), f32),
                   jax.ShapeDtypeStruct((N_P, 16), f32)],
    )(x_p, W_att, ASD)


def _comb1_body(a0_ref, a1_ref, b0_ref, b1_ref, rs_ref, rd_ref, w_ref,
                h_ref, nrm_ref, g_ref):
    num = a0_ref[...] + a1_ref[...]
    tails = b0_ref[...] + b1_ref[...]
    sexp_b = jnp.dot(tails, rs_ref[...], preferred_element_type=f32)
    deg_b = jnp.dot(tails, rd_ref[...], preferred_element_type=f32) + 1.0
    z = num / (sexp_b + 1e-16)
    rid = pl.program_id(0) * TBLK + lax.broadcasted_iota(i32, (TBLK, D), 0)
    h = jnp.where(rid < N, jnp.maximum(z, 0.0), 0.0)
    nrm = lax.rsqrt(deg_b)
    h_ref[...] = h
    nrm_ref[...] = nrm
    g_ref[...] = jnp.dot(h, w_ref[...], preferred_element_type=f32) * nrm


def _tc_comb1(acc0, acc1, t0, t1, RS, RD, W1):
    return pl.pallas_call(
        _comb1_body,
        grid=(TGRID,),
        in_specs=[pl.BlockSpec((TBLK, D), lambda i: (i, 0)),
                  pl.BlockSpec((TBLK, D), lambda i: (i, 0)),
                  pl.BlockSpec((TBLK, 16), lambda i: (i, 0)),
                  pl.BlockSpec((TBLK, 16), lambda i: (i, 0)),
                  pl.BlockSpec((16, D), lambda i: (0, 0)),
                  pl.BlockSpec((16, D), lambda i: (0, 0)),
                  pl.BlockSpec((D, D), lambda i: (0, 0))],
        out_specs=[pl.BlockSpec((TBLK, D), lambda i: (i, 0)),
                   pl.BlockSpec((TBLK, D), lambda i: (i, 0)),
                   pl.BlockSpec((TBLK, D), lambda i: (i, 0))],
        out_shape=[jax.ShapeDtypeStruct((N_P, D), f32),
                   jax.ShapeDtypeStruct((N_P, D), f32),
                   jax.ShapeDtypeStruct((N_P, D), f32)],
    )(acc0, acc1, t0, t1, RS, RD, W1)


def _layer_body(h_ref, g_ref, nrm_ref, a0_ref, a1_ref, w_ref, b_ref,
                hn_ref, gn_ref):
    nrm = nrm_ref[...]
    o = nrm * (a0_ref[...] + a1_ref[...] + g_ref[...]) + b_ref[...]
    rid = pl.program_id(0) * TBLK + lax.broadcasted_iota(i32, (TBLK, D), 0)
    hn = h_ref[...] + jnp.where(rid < N, jnp.maximum(o, 0.0), 0.0)
    hn_ref[...] = hn
    gn_ref[...] = jnp.dot(hn, w_ref[...], preferred_element_type=f32) * nrm


def _tc_layer(h, g, nrm, acc0, acc1, Wn, b):
    return pl.pallas_call(
        _layer_body,
        grid=(TGRID,),
        in_specs=[pl.BlockSpec((TBLK, D), lambda i: (i, 0)),
                  pl.BlockSpec((TBLK, D), lambda i: (i, 0)),
                  pl.BlockSpec((TBLK, D), lambda i: (i, 0)),
                  pl.BlockSpec((TBLK, D), lambda i: (i, 0)),
                  pl.BlockSpec((TBLK, D), lambda i: (i, 0)),
                  pl.BlockSpec((D, D), lambda i: (0, 0)),
                  pl.BlockSpec((1, D), lambda i: (0, 0))],
        out_specs=[pl.BlockSpec((TBLK, D), lambda i: (i, 0)),
                   pl.BlockSpec((TBLK, D), lambda i: (i, 0))],
        out_shape=[jax.ShapeDtypeStruct((N_P, D), f32),
                   jax.ShapeDtypeStruct((N_P, D), f32)],
    )(h, g, nrm, acc0, acc1, Wn, b)


def _final_body(g_ref, nrm_ref, a0_ref, a1_ref, b_ref, o_ref):
    o_ref[...] = (nrm_ref[...] * (a0_ref[...] + a1_ref[...] + g_ref[...])
                  + b_ref[...])


def _tc_final(g, nrm, acc0, acc1, b):
    return pl.pallas_call(
        _final_body,
        grid=(TGRID,),
        in_specs=[pl.BlockSpec((TBLK, D), lambda i: (i, 0)),
                  pl.BlockSpec((TBLK, D), lambda i: (i, 0)),
                  pl.BlockSpec((TBLK, D), lambda i: (i, 0)),
                  pl.BlockSpec((TBLK, D), lambda i: (i, 0)),
                  pl.BlockSpec((1, D), lambda i: (0, 0))],
        out_specs=pl.BlockSpec((TBLK, D), lambda i: (i, 0)),
        out_shape=jax.ShapeDtypeStruct((N_P, D), f32),
    )(g, nrm, acc0, acc1, b)


_SC_MESH = plsc.VectorSubcoreMesh(core_axis_name="c", subcore_axis_name="s",
                                  num_cores=NC, num_subcores=NS)


def _acc_init(zr_hbm, bounce, acc, s, cb):
    pltpu.sync_copy(zr_hbm.at[pl.ds(0, cb)], bounce)
    base = s * RPT
    nfull, rem = divmod(RPT, cb)
    for k in range(nfull):
        pltpu.sync_copy(bounce, acc.at[pl.ds(base + k * cb, cb)])
    if rem:
        pltpu.sync_copy(bounce.at[pl.ds(0, rem)],
                        acc.at[pl.ds(base + nfull * cb, rem)])
    plsc.subcore_barrier()


def _acc_writeback(acc, bounce, out, c, s, cb):
    plsc.subcore_barrier()
    base = s * RPT
    nfull, rem = divmod(RPT, cb)
    for k in range(nfull):
        r0 = base + k * cb
        pltpu.sync_copy(acc.at[pl.ds(r0, cb)], bounce)
        pltpu.sync_copy(bounce, out.at[c, pl.ds(r0, cb)])
    if rem:
        r0 = base + nfull * cb
        pltpu.sync_copy(acc.at[pl.ds(r0, rem)], bounce.at[pl.ds(0, rem)])
        pltpu.sync_copy(bounce.at[pl.ds(0, rem)], out.at[c, pl.ds(r0, rem)])


def _edge_w(srows, drows, i):
    e = srows[i] + drows[i]
    e = jnp.maximum(e, 0.2 * e)
    return jnp.exp(e)


def _att_edges(rows, srows, drows):
    def edge(i, carry2):
        w = _edge_w(srows, drows, i)
        for h in range(HEAD):
            wv = jnp.full((LANES,), w[h], f32)
            for k2 in range(2):
                col = h * DH + k2 * LANES
                rows[i, pl.ds(col, LANES)] = rows[i, pl.ds(col, LANES)] * wv
        return carry2

    lax.fori_loop(0, BBA, edge, 0)


def _attf_sc_body(srcp, dstp, zh_t, als_t, ald_t, zr_hbm, out,
                  ra, rb, sra, srb, dra, drb, gia, gib, dia, dib, acc, sem):
    c = lax.axis_index("c")
    s = lax.axis_index("s")
    wid = c * NS + s
    _acc_init(zr_hbm, ra, acc, s, BBA)

    def pair(p, carry):
        b0 = (2 * p) * BBA
        b1 = b0 + BBA
        i1 = pltpu.async_copy(srcp.at[wid, pl.ds(b0, BBA)], gia, sem)
        i2 = pltpu.async_copy(dstp.at[wid, pl.ds(b0, BBA)], dia, sem)
        i3 = pltpu.async_copy(srcp.at[wid, pl.ds(b1, BBA)], gib, sem)
        i4 = pltpu.async_copy(dstp.at[wid, pl.ds(b1, BBA)], dib, sem)
        i1.wait(); i2.wait()
        g1 = pltpu.async_copy(als_t.at[gia], sra, sem)
        g2 = pltpu.async_copy(ald_t.at[dia], dra, sem)
        g3 = pltpu.async_copy(zh_t.at[gia], ra, sem)
        i3.wait(); i4.wait()
        g4 = pltpu.async_copy(als_t.at[gib], srb, sem)
        g5 = pltpu.async_copy(ald_t.at[dib], drb, sem)
        g6 = pltpu.async_copy(zh_t.at[gib], rb, sem)
        g1.wait(); g2.wait(); g3.wait()
        _att_edges(ra, sra, dra)
        pltpu.sync_copy(ra, acc.at[dia], add=True)
        g4.wait(); g5.wait(); g6.wait()
        _att_edges(rb, srb, drb)
        pltpu.sync_copy(rb, acc.at[dib], add=True)
        return carry

    lax.fori_loop(0, NBA // 2, pair, 0)
    _acc_writeback(acc, ra, out, c, s, BBA)


_attf_sc = pl.kernel(
    _attf_sc_body,
    out_type=jax.ShapeDtypeStruct((NC, ACC_R, D), f32),
    mesh=_SC_MESH,
    compiler_params=pltpu.CompilerParams(use_tc_tiling_on_sc=False),
    scratch_types=[
        pltpu.VMEM((BBA, D), f32),
        pltpu.VMEM((BBA, D), f32),
        pltpu.VMEM((BBA, LANES), f32),
        pltpu.VMEM((BBA, LANES), f32),
        pltpu.VMEM((BBA, LANES), f32),
        pltpu.VMEM((BBA, LANES), f32),
        pltpu.VMEM((BBA,), i32),
        pltpu.VMEM((BBA,), i32),
        pltpu.VMEM((BBA,), i32),
        pltpu.VMEM((BBA,), i32),
        pltpu.VMEM_SHARED((ACC_R, D), f32),
        pltpu.SemaphoreType.DMA,
    ],
)


def _attt_sc_body(srcp, dstp, als_t, ald_t, zr16_hbm, out,
                  ta, tb, sra, srb, dra, drb, gia, gib, dia, dib, acc, sem):
    c = lax.axis_index("c")
    s = lax.axis_index("s")
    wid = c * NS + s
    _acc_init(zr16_hbm, ta, acc, s, BBA)
    iot = lax.iota(i32, LANES)
    tail_one = jnp.where(iot == HEAD, 1.0, 0.0).astype(f32)

    def tails(trows, srows, drows):
        def edge(i, carry2):
            w = _edge_w(srows, drows, i)
            trows[i] = jnp.where(iot < HEAD, w, tail_one)
            return carry2

        lax.fori_loop(0, BBA, edge, 0)

    def pair(p, carry):
        b0 = (2 * p) * BBA
        b1 = b0 + BBA
        i1 = pltpu.async_copy(srcp.at[wid, pl.ds(b0, BBA)], gia, sem)
        i2 = pltpu.async_copy(dstp.at[wid, pl.ds(b0, BBA)], dia, sem)
        i3 = pltpu.async_copy(srcp.at[wid, pl.ds(b1, BBA)], gib, sem)
        i4 = pltpu.async_copy(dstp.at[wid, pl.ds(b1, BBA)], dib, sem)
        i1.wait(); i2.wait()
        g1 = pltpu.async_copy(als_t.at[gia], sra, sem)
        g2 = pltpu.async_copy(ald_t.at[dia], dra, sem)
        i3.wait(); i4.wait()
        g3 = pltpu.async_copy(als_t.at[gib], srb, sem)
        g4 = pltpu.async_copy(ald_t.at[dib], drb, sem)
        g1.wait(); g2.wait()
        tails(ta, sra, dra)
        pltpu.sync_copy(ta, acc.at[dia], add=True)
        g3.wait(); g4.wait()
        tails(tb, srb, drb)
        pltpu.sync_copy(tb, acc.at[dib], add=True)
        return carry

    lax.fori_loop(0, NBA // 2, pair, 0)
    _acc_writeback(acc, ta, out, c, s, BBA)


_attt_sc = pl.kernel(
    _attt_sc_body,
    out_type=jax.ShapeDtypeStruct((NC, ACC_R, LANES), f32),
    mesh=_SC_MESH,
    compiler_params=pltpu.CompilerParams(use_tc_tiling_on_sc=False),
    scratch_types=[
        pltpu.VMEM((BBA, LANES), f32),
        pltpu.VMEM((BBA, LANES), f32),
        pltpu.VMEM((BBA, LANES), f32),
        pltpu.VMEM((BBA, LANES), f32),
        pltpu.VMEM((BBA, LANES), f32),
        pltpu.VMEM((BBA, LANES), f32),
        pltpu.VMEM((BBA,), i32),
        pltpu.VMEM((BBA,), i32),
        pltpu.VMEM((BBA,), i32),
        pltpu.VMEM((BBA,), i32),
        pltpu.VMEM_SHARED((ACC_R, LANES), f32),
        pltpu.SemaphoreType.DMA,
    ],
)


def _gcn_sc_body(srcp3, dstp3, g_t, zr_hbm, out, idx_s, dia, dib, ra, rb,
                 acc, sem):
    c = lax.axis_index("c")
    s = lax.axis_index("s")
    wid = c * NS + s
    _acc_init(zr_hbm, ra, acc, s, BB)
    pltpu.sync_copy(srcp3.at[wid], idx_s)

    def pair(p, carry):
        j0 = 2 * p
        j1 = j0 + 1
        ga = pltpu.async_copy(g_t.at[idx_s.at[j0]], ra, sem)
        da = pltpu.async_copy(dstp3.at[wid, j0], dia, sem)
        gb = pltpu.async_copy(g_t.at[idx_s.at[j1]], rb, sem)
        db = pltpu.async_copy(dstp3.at[wid, j1], dib, sem)
        ga.wait(); da.wait()
        pltpu.sync_copy(ra, acc.at[dia], add=True)
        gb.wait(); db.wait()
        pltpu.sync_copy(rb, acc.at[dib], add=True)
        return carry

    lax.fori_loop(0, NB // 2, pair, 0)
    _acc_writeback(acc, ra, out, c, s, BB)


_gcn_sc = pl.kernel(
    _gcn_sc_body,
    out_type=jax.ShapeDtypeStruct((NC, ACC_R, D), f32),
    mesh=_SC_MESH,
    compiler_params=pltpu.CompilerParams(use_tc_tiling_on_sc=False),
    scratch_types=[
        pltpu.VMEM((NB, BB), i32),
        pltpu.VMEM((BB,), i32),
        pltpu.VMEM((BB,), i32),
        pltpu.VMEM((BB, D), f32),
        pltpu.VMEM((BB, D), f32),
        pltpu.VMEM_SHARED((ACC_R, D), f32),
        pltpu.SemaphoreType.DMA,
    ],
)


def kernel(x, edge_index, W_att, a_src, a_dst,
           W1, b1, W2, b2, W3, b3, W4, b4, W5, b5):
    x_p = jnp.concatenate([x, jnp.zeros((N_P - N, D), f32)], axis=0)
    padt = jnp.full((NW, EROW - EPT), N, i32)
    src2 = jnp.concatenate(
        [jnp.concatenate([edge_index[0], jnp.full((E_P - E,), N, i32)])
         .reshape(NW, EPT), padt], axis=1)
    dst2 = jnp.concatenate(
        [jnp.concatenate([edge_index[1], jnp.full((E_P - E,), N, i32)])
         .reshape(NW, EPT), padt], axis=1)
    src3 = src2[:, :EPT].reshape(NW, NB, BB)
    dst3 = dst2[:, :EPT].reshape(NW, NB, BB)

    karr = jnp.arange(D)
    h16 = jnp.arange(16)
    mask_s = ((karr[:, None] // DH == h16[None, :]) &
              (h16[None, :] < HEAD)).astype(f32)
    mask_d = ((karr[:, None] // DH == h16[None, :] - HEAD) &
              (h16[None, :] >= HEAD) & (h16[None, :] < 2 * HEAD)).astype(f32)
    ASD = (a_src.reshape(D)[:, None] * mask_s
           + a_dst.reshape(D)[:, None] * mask_d)
    RS = ((h16[:, None] < HEAD) &
          (h16[:, None] == karr[None, :] // DH)).astype(f32)
    RD = (h16[:, None] == HEAD).astype(f32) * jnp.ones((1, D), f32)

    zh_t, alsd16 = _tc_pre(x_p, W_att, ASD)
    als_t = alsd16
    ald_t = jnp.roll(alsd16, -HEAD, axis=1)

    zr = jnp.zeros((BB, D), f32)
    zr16 = jnp.zeros((BBA, 16), f32)
    apad = jnp.zeros((N_P - ACC_R, D), f32)
    apad16 = jnp.zeros((N_P - ACC_R, 16), f32)

    def padacc(a):
        return jnp.concatenate([a, apad], axis=0)

    def padacc16(a):
        return jnp.concatenate([a, apad16], axis=0)

    accf = _attf_sc(src2, dst2, zh_t, als_t, ald_t, zr)
    acct = _attt_sc(src2, dst2, als_t, ald_t, zr16)
    h, nrm, g = _tc_comb1(padacc(accf[0]), padacc(accf[1]),
                          padacc16(acct[0]), padacc16(acct[1]), RS, RD, W1)

    for Wn, b in ((W2, b1), (W3, b2), (W4, b3), (W5, b4)):
        accg = _gcn_sc(src3, dst3, g, zr)
        h, g = _tc_layer(h, g, nrm, padacc(accg[0]), padacc(accg[1]),
                         Wn, b.reshape(1, D))
    accg = _gcn_sc(src3, dst3, g, zr)
    out = _tc_final(g, nrm, padacc(accg[0]), padacc(accg[1]),
                    b5.reshape(1, D))
    return out[:N]


# speedup vs baseline: 1.0998x; 1.0637x over previous
import jax
import jax.numpy as jnp
from jax import lax
from jax.experimental import pallas as pl
from jax.experimental.pallas import tpu as pltpu
from jax.experimental.pallas import tpu_sc as plsc

N = 10000
E = 320000
D = 128
HEAD = 4
DH = 32

NC = 2
NS = 16
NW = NC * NS
LANES = 16

N_P = 10240
ACC_R = 10016
BB = 128
NB = 80
BBA = 80
NBA = 128
EPT = NB * BB
E_P = NW * EPT
EROW = EPT + 80
TBLK = 256
TGRID = N_P // TBLK
RPT = ACC_R // NS

f32 = jnp.float32
i32 = jnp.int32


def _pre_body(x_ref, w_ref, asd_ref, zh_ref, alsd_ref):
    z = jnp.dot(x_ref[...], w_ref[...], preferred_element_type=f32)
    zh_ref[...] = z
    alsd_ref[...] = jnp.dot(z, asd_ref[...], preferred_element_type=f32)


def _tc_pre(x_p, W_att, ASD):
    return pl.pallas_call(
        _pre_body,
        grid=(TGRID,),
        in_specs=[pl.BlockSpec((TBLK, D), lambda i: (i, 0)),
                  pl.BlockSpec((D, D), lambda i: (0, 0)),
                  pl.BlockSpec((D, 16), lambda i: (0, 0))],
        out_specs=[pl.BlockSpec((TBLK, D), lambda i: (i, 0)),
                   pl.BlockSpec((TBLK, 16), lambda i: (i, 0))],
        out_shape=[jax.ShapeDtypeStruct((N_P, D), f32),
                   jax.ShapeDtypeStruct((N_P, 16), f32)],
    )(x_p, W_att, ASD)


def _comb1_body(a0_ref, a1_ref, b0_ref, b1_ref, rs_ref, rd_ref, w_ref,
                h_ref, nrm_ref, g_ref):
    num = a0_ref[...] + a1_ref[...]
    tails = b0_ref[...] + b1_ref[...]
    sexp_b = jnp.dot(tails, rs_ref[...], preferred_element_type=f32)
    deg_b = jnp.dot(tails, rd_ref[...], preferred_element_type=f32) + 1.0
    z = num / (sexp_b + 1e-16)
    rid = pl.program_id(0) * TBLK + lax.broadcasted_iota(i32, (TBLK, D), 0)
    h = jnp.where(rid < N, jnp.maximum(z, 0.0), 0.0)
    nrm = lax.rsqrt(deg_b)
    h_ref[...] = h
    nrm_ref[...] = nrm
    g_ref[...] = jnp.dot(h, w_ref[...], preferred_element_type=f32) * nrm


def _tc_comb1(acc0, acc1, t0, t1, RS, RD, W1):
    return pl.pallas_call(
        _comb1_body,
        grid=(TGRID,),
        in_specs=[pl.BlockSpec((TBLK, D), lambda i: (i, 0)),
                  pl.BlockSpec((TBLK, D), lambda i: (i, 0)),
                  pl.BlockSpec((TBLK, 16), lambda i: (i, 0)),
                  pl.BlockSpec((TBLK, 16), lambda i: (i, 0)),
                  pl.BlockSpec((16, D), lambda i: (0, 0)),
                  pl.BlockSpec((16, D), lambda i: (0, 0)),
                  pl.BlockSpec((D, D), lambda i: (0, 0))],
        out_specs=[pl.BlockSpec((TBLK, D), lambda i: (i, 0)),
                   pl.BlockSpec((TBLK, D), lambda i: (i, 0)),
                   pl.BlockSpec((TBLK, D), lambda i: (i, 0))],
        out_shape=[jax.ShapeDtypeStruct((N_P, D), f32),
                   jax.ShapeDtypeStruct((N_P, D), f32),
                   jax.ShapeDtypeStruct((N_P, D), f32)],
    )(acc0, acc1, t0, t1, RS, RD, W1)


def _layer_body(h_ref, g_ref, nrm_ref, a0_ref, a1_ref, w_ref, b_ref,
                hn_ref, gn_ref):
    nrm = nrm_ref[...]
    o = nrm * (a0_ref[...] + a1_ref[...] + g_ref[...]) + b_ref[...]
    rid = pl.program_id(0) * TBLK + lax.broadcasted_iota(i32, (TBLK, D), 0)
    hn = h_ref[...] + jnp.where(rid < N, jnp.maximum(o, 0.0), 0.0)
    hn_ref[...] = hn
    gn_ref[...] = jnp.dot(hn, w_ref[...], preferred_element_type=f32) * nrm


def _tc_layer(h, g, nrm, acc0, acc1, Wn, b):
    return pl.pallas_call(
        _layer_body,
        grid=(TGRID,),
        in_specs=[pl.BlockSpec((TBLK, D), lambda i: (i, 0)),
                  pl.BlockSpec((TBLK, D), lambda i: (i, 0)),
                  pl.BlockSpec((TBLK, D), lambda i: (i, 0)),
                  pl.BlockSpec((TBLK, D), lambda i: (i, 0)),
                  pl.BlockSpec((TBLK, D), lambda i: (i, 0)),
                  pl.BlockSpec((D, D), lambda i: (0, 0)),
                  pl.BlockSpec((1, D), lambda i: (0, 0))],
        out_specs=[pl.BlockSpec((TBLK, D), lambda i: (i, 0)),
                   pl.BlockSpec((TBLK, D), lambda i: (i, 0))],
        out_shape=[jax.ShapeDtypeStruct((N_P, D), f32),
                   jax.ShapeDtypeStruct((N_P, D), f32)],
    )(h, g, nrm, acc0, acc1, Wn, b)


def _final_body(g_ref, nrm_ref, a0_ref, a1_ref, b_ref, o_ref):
    o_ref[...] = (nrm_ref[...] * (a0_ref[...] + a1_ref[...] + g_ref[...])
                  + b_ref[...])


def _tc_final(g, nrm, acc0, acc1, b):
    return pl.pallas_call(
        _final_body,
        grid=(TGRID,),
        in_specs=[pl.BlockSpec((TBLK, D), lambda i: (i, 0)),
                  pl.BlockSpec((TBLK, D), lambda i: (i, 0)),
                  pl.BlockSpec((TBLK, D), lambda i: (i, 0)),
                  pl.BlockSpec((TBLK, D), lambda i: (i, 0)),
                  pl.BlockSpec((1, D), lambda i: (0, 0))],
        out_specs=pl.BlockSpec((TBLK, D), lambda i: (i, 0)),
        out_shape=jax.ShapeDtypeStruct((N_P, D), f32),
    )(g, nrm, acc0, acc1, b)


_SC_MESH = plsc.VectorSubcoreMesh(core_axis_name="c", subcore_axis_name="s",
                                  num_cores=NC, num_subcores=NS)


def _acc_init(zr_hbm, bounce, acc, s, cb):
    pltpu.sync_copy(zr_hbm.at[pl.ds(0, cb)], bounce)
    base = s * RPT
    nfull, rem = divmod(RPT, cb)
    for k in range(nfull):
        pltpu.sync_copy(bounce, acc.at[pl.ds(base + k * cb, cb)])
    if rem:
        pltpu.sync_copy(bounce.at[pl.ds(0, rem)],
                        acc.at[pl.ds(base + nfull * cb, rem)])
    plsc.subcore_barrier()


def _acc_writeback(acc, bounce, out, c, s, cb):
    plsc.subcore_barrier()
    base = s * RPT
    nfull, rem = divmod(RPT, cb)
    for k in range(nfull):
        r0 = base + k * cb
        pltpu.sync_copy(acc.at[pl.ds(r0, cb)], bounce)
        pltpu.sync_copy(bounce, out.at[c, pl.ds(r0, cb)])
    if rem:
        r0 = base + nfull * cb
        pltpu.sync_copy(acc.at[pl.ds(r0, rem)], bounce.at[pl.ds(0, rem)])
        pltpu.sync_copy(bounce.at[pl.ds(0, rem)], out.at[c, pl.ds(r0, rem)])


def _edge_w(srows, drows, i):
    e = srows[i] + drows[i]
    e = jnp.maximum(e, 0.2 * e)
    return jnp.exp(e)


def _attm_sc_body(srcp, dstp, zh_t, als_t, ald_t, zr_hbm, zr16_hbm,
                  outf, outt,
                  ra, rb, ta, tb, sra, srb, dra, drb, gia, gib, dia, dib,
                  accf, acct, sem):
    c = lax.axis_index("c")
    s = lax.axis_index("s")
    wid = c * NS + s
    _acc_init(zr_hbm, ra, accf, s, BBA)
    _acc_init(zr16_hbm, ta, acct, s, BBA)
    iot = lax.iota(i32, LANES)
    tail_one = jnp.where(iot == HEAD, 1.0, 0.0).astype(f32)

    def edges(rows, trows, srows, drows):
        def edge(i, carry2):
            w = _edge_w(srows, drows, i)
            trows[i] = jnp.where(iot < HEAD, w, tail_one)
            for h in range(HEAD):
                wv = jnp.full((LANES,), w[h], f32)
                for k2 in range(2):
                    col = h * DH + k2 * LANES
                    rows[i, pl.ds(col, LANES)] = (rows[i, pl.ds(col, LANES)]
                                                  * wv)
            return carry2

        lax.fori_loop(0, BBA, edge, 0)

    def pair(p, carry):
        b0 = (2 * p) * BBA
        b1 = b0 + BBA
        i1 = pltpu.async_copy(srcp.at[wid, pl.ds(b0, BBA)], gia, sem)
        i2 = pltpu.async_copy(dstp.at[wid, pl.ds(b0, BBA)], dia, sem)
        i3 = pltpu.async_copy(srcp.at[wid, pl.ds(b1, BBA)], gib, sem)
        i4 = pltpu.async_copy(dstp.at[wid, pl.ds(b1, BBA)], dib, sem)
        i1.wait(); i2.wait()
        g1 = pltpu.async_copy(als_t.at[gia], sra, sem)
        g2 = pltpu.async_copy(ald_t.at[dia], dra, sem)
        g3 = pltpu.async_copy(zh_t.at[gia], ra, sem)
        i3.wait(); i4.wait()
        g4 = pltpu.async_copy(als_t.at[gib], srb, sem)
        g5 = pltpu.async_copy(ald_t.at[dib], drb, sem)
        g6 = pltpu.async_copy(zh_t.at[gib], rb, sem)
        g1.wait(); g2.wait(); g3.wait()
        edges(ra, ta, sra, dra)
        pltpu.sync_copy(ra, accf.at[dia], add=True)
        pltpu.sync_copy(ta, acct.at[dia], add=True)
        g4.wait(); g5.wait(); g6.wait()
        edges(rb, tb, srb, drb)
        pltpu.sync_copy(rb, accf.at[dib], add=True)
        pltpu.sync_copy(tb, acct.at[dib], add=True)
        return carry

    lax.fori_loop(0, NBA // 2, pair, 0)
    _acc_writeback(accf, ra, outf, c, s, BBA)
    _acc_writeback(acct, ta, outt, c, s, BBA)


_attm_sc = pl.kernel(
    _attm_sc_body,
    out_type=[jax.ShapeDtypeStruct((NC, ACC_R, D), f32),
              jax.ShapeDtypeStruct((NC, ACC_R, LANES), f32)],
    mesh=_SC_MESH,
    compiler_params=pltpu.CompilerParams(use_tc_tiling_on_sc=False),
    scratch_types=[
        pltpu.VMEM((BBA, D), f32),
        pltpu.VMEM((BBA, D), f32),
        pltpu.VMEM((BBA, LANES), f32),
        pltpu.VMEM((BBA, LANES), f32),
        pltpu.VMEM((BBA, LANES), f32),
        pltpu.VMEM((BBA, LANES), f32),
        pltpu.VMEM((BBA, LANES), f32),
        pltpu.VMEM((BBA, LANES), f32),
        pltpu.VMEM((BBA,), i32),
        pltpu.VMEM((BBA,), i32),
        pltpu.VMEM((BBA,), i32),
        pltpu.VMEM((BBA,), i32),
        pltpu.VMEM_SHARED((ACC_R, D), f32),
        pltpu.VMEM_SHARED((ACC_R, LANES), f32),
        pltpu.SemaphoreType.DMA,
    ],
)


def _gcn_sc_body(srcp3, dstp3, g_t, zr_hbm, out, idx_s, dia, dib, ra, rb,
                 acc, sem):
    c = lax.axis_index("c")
    s = lax.axis_index("s")
    wid = c * NS + s
    _acc_init(zr_hbm, ra, acc, s, BB)
    pltpu.sync_copy(srcp3.at[wid], idx_s)

    def pair(p, carry):
        j0 = 2 * p
        j1 = j0 + 1
        ga = pltpu.async_copy(g_t.at[idx_s.at[j0]], ra, sem)
        da = pltpu.async_copy(dstp3.at[wid, j0], dia, sem)
        gb = pltpu.async_copy(g_t.at[idx_s.at[j1]], rb, sem)
        db = pltpu.async_copy(dstp3.at[wid, j1], dib, sem)
        ga.wait(); da.wait()
        pltpu.sync_copy(ra, acc.at[dia], add=True)
        gb.wait(); db.wait()
        pltpu.sync_copy(rb, acc.at[dib], add=True)
        return carry

    lax.fori_loop(0, NB // 2, pair, 0)
    _acc_writeback(acc, ra, out, c, s, BB)


_gcn_sc = pl.kernel(
    _gcn_sc_body,
    out_type=jax.ShapeDtypeStruct((NC, ACC_R, D), f32),
    mesh=_SC_MESH,
    compiler_params=pltpu.CompilerParams(use_tc_tiling_on_sc=False),
    scratch_types=[
        pltpu.VMEM((NB, BB), i32),
        pltpu.VMEM((BB,), i32),
        pltpu.VMEM((BB,), i32),
        pltpu.VMEM((BB, D), f32),
        pltpu.VMEM((BB, D), f32),
        pltpu.VMEM_SHARED((ACC_R, D), f32),
        pltpu.SemaphoreType.DMA,
    ],
)


def kernel(x, edge_index, W_att, a_src, a_dst,
           W1, b1, W2, b2, W3, b3, W4, b4, W5, b5):
    x_p = jnp.concatenate([x, jnp.zeros((N_P - N, D), f32)], axis=0)
    padt = jnp.full((NW, EROW - EPT), N, i32)
    src2 = jnp.concatenate(
        [jnp.concatenate([edge_index[0], jnp.full((E_P - E,), N, i32)])
         .reshape(NW, EPT), padt], axis=1)
    dst2 = jnp.concatenate(
        [jnp.concatenate([edge_index[1], jnp.full((E_P - E,), N, i32)])
         .reshape(NW, EPT), padt], axis=1)
    src3 = src2[:, :EPT].reshape(NW, NB, BB)
    dst3 = dst2[:, :EPT].reshape(NW, NB, BB)

    karr = jnp.arange(D)
    h16 = jnp.arange(16)
    mask_s = ((karr[:, None] // DH == h16[None, :]) &
              (h16[None, :] < HEAD)).astype(f32)
    mask_d = ((karr[:, None] // DH == h16[None, :] - HEAD) &
              (h16[None, :] >= HEAD) & (h16[None, :] < 2 * HEAD)).astype(f32)
    ASD = (a_src.reshape(D)[:, None] * mask_s
           + a_dst.reshape(D)[:, None] * mask_d)
    RS = ((h16[:, None] < HEAD) &
          (h16[:, None] == karr[None, :] // DH)).astype(f32)
    RD = (h16[:, None] == HEAD).astype(f32) * jnp.ones((1, D), f32)

    zh_t, alsd16 = _tc_pre(x_p, W_att, ASD)
    als_t = alsd16
    ald_t = jnp.roll(alsd16, -HEAD, axis=1)

    zr = jnp.zeros((BB, D), f32)
    zr16 = jnp.zeros((BBA, 16), f32)
    apad = jnp.zeros((N_P - ACC_R, D), f32)
    apad16 = jnp.zeros((N_P - ACC_R, 16), f32)

    def padacc(a):
        return jnp.concatenate([a, apad], axis=0)

    def padacc16(a):
        return jnp.concatenate([a, apad16], axis=0)

    accf, acct = _attm_sc(src2, dst2, zh_t, als_t, ald_t, zr, zr16)
    h, nrm, g = _tc_comb1(padacc(accf[0]), padacc(accf[1]),
                          padacc16(acct[0]), padacc16(acct[1]), RS, RD, W1)

    for Wn, b in ((W2, b1), (W3, b2), (W4, b3), (W5, b4)):
        accg = _gcn_sc(src3, dst3, g, zr)
        h, g = _tc_layer(h, g, nrm, padacc(accg[0]), padacc(accg[1]),
                         Wn, b.reshape(1, D))
    accg = _gcn_sc(src3, dst3, g, zr)
    out = _tc_final(g, nrm, padacc(accg[0]), padacc(accg[1]),
                    b5.reshape(1, D))
    return out[:N]
